# baseline (speedup 1.0000x reference)
import math

import jax
import jax.numpy as jnp
from jax import lax
from jax.experimental import pallas as pl
from jax.experimental.pallas import tpu as pltpu
from jax.experimental.pallas import tpu_sc as plsc

N = 10000
E = 320000
D = 128

NC = 2
NS = 16
NW = NC * NS
EW = E // NW
CH = 80
C = EW // CH
G = CH // 16
INV_SQRT_D = 1.0 / math.sqrt(float(D))


def _k1_body(x_ref, w1, b1, w2, b2, w3, b3, w4, b4, w5, b5,
             base_ref, h2_ref, h3_ref, h4_ref, a_ref, c_ref):
    xb = x_ref[...]
    dot = lambda m, w: lax.dot_general(
        m, w[...], (((1,), (1,)), ((), ())), preferred_element_type=jnp.float32)
    base_ref[...] = dot(xb, w1) + b1[...][None, :]
    h2_ref[...] = dot(xb, w2) + b2[...][None, :]
    h3 = dot(xb, w3) + b3[...][None, :]
    h3_ref[...] = h3.astype(jnp.bfloat16)
    h4_ref[...] = (dot(xb, w4) + b4[...][None, :]).astype(jnp.bfloat16)
    a_ref[...] = jnp.sum(h3 * w5[...][None, :], axis=1, keepdims=True)
    c_ref[...] = jnp.sum(h3 * b5[...][None, :], axis=1, keepdims=True)


def _node_transforms(x, W1, b1, W2, b2, W3, b3, W4, b4, w5, b5):
    BN = 1000
    row_spec = pl.BlockSpec((BN, D), lambda i: (i, 0))
    w_spec = pl.BlockSpec((D, D), lambda i: (0, 0))
    b_spec = pl.BlockSpec((D,), lambda i: (0,))
    v_spec = pl.BlockSpec((BN, 1), lambda i: (i, 0))
    return pl.pallas_call(
        _k1_body,
        grid=(N // BN,),
        in_specs=[row_spec, w_spec, b_spec, w_spec, b_spec, w_spec, b_spec,
                  w_spec, b_spec, b_spec, b_spec],
        out_specs=[row_spec, row_spec, row_spec, row_spec, v_spec, v_spec],
        out_shape=[
            jax.ShapeDtypeStruct((N, D), jnp.float32),
            jax.ShapeDtypeStruct((N, D), jnp.float32),
            jax.ShapeDtypeStruct((N, D), jnp.bfloat16),
            jax.ShapeDtypeStruct((N, D), jnp.bfloat16),
            jax.ShapeDtypeStruct((N, 1), jnp.float32),
            jax.ShapeDtypeStruct((N, 1), jnp.float32),
        ],
    )(x, W1, b1, W2, b2, W3, b3, W4, b4, w5, b5)


def _sc_mesh():
    return plsc.VectorSubcoreMesh(
        core_axis_name="c", subcore_axis_name="s", num_cores=NC,
        num_subcores=NS)


def _edge_scores_call(src3, dst3, dst2, ea2, h3, h4, a, c):
    def body(src_hbm, dst_hbm, dstf_hbm, ea_hbm, h3_hbm, h4_hbm, a_hbm, c_hbm,
             ex_hbm, dpart_hbm,
             srci, dsti, eav, exrow, av, cv, h3b, h4b, pbuf, dshared,
             s3a, s3b, s3c, s4a, s4b, s4c, sexa, sexb, sexc,
             ssca, sscb, sscc):
        iota16 = lax.iota(jnp.int32, 16)
        cid = lax.axis_index("c")
        sid = lax.axis_index("s")
        wid = sid * NC + cid
        s3 = (s3a, s3b, s3c)
        s4 = (s4a, s4b, s4c)
        sex = (sexa, sexb, sexc)
        ssc = (ssca, sscb, sscc)

        @pl.when(sid == 0)
        def _():
            def zb(i, carry):
                av[pl.ds(i * 16, 16)] = jnp.zeros((16,), jnp.float32)
                return carry
            lax.fori_loop(0, N // 16, zb, 0)
            pltpu.sync_copy(av, dshared)

        pltpu.sync_copy(src_hbm.at[wid], srci)
        pltpu.sync_copy(dst_hbm.at[wid], dsti)
        pltpu.sync_copy(ea_hbm.at[wid], eav)
        pltpu.sync_copy(a_hbm, av)
        pltpu.sync_copy(c_hbm, cv)
        plsc.subcore_barrier()

        def issue_gathers(i, b):
            pltpu.async_copy(h3_hbm.at[dsti.at[i]], h3b.at[b], s3[b])
            pltpu.async_copy(h4_hbm.at[srci.at[i]], h4b.at[b], s4[b])

        def wait_gathers(i, b):
            pltpu.make_async_copy(h3_hbm.at[dsti.at[i]], h3b.at[b],
                                  s3[b]).wait()
            pltpu.make_async_copy(h4_hbm.at[srci.at[i]], h4b.at[b],
                                  s4[b]).wait()

        def compute_chunk(i, b):
            def grp(g, carry):
                for j in range(16):
                    r = g * 16 + j
                    p = jnp.zeros((16,), jnp.float32)
                    q = jnp.zeros((16,), jnp.float32)
                    for k in range(4):
                        va = plsc.bitcast(h3b[b, r, pl.ds(k * 16, 16)],
                                          jnp.bfloat16)
                        vb = plsc.bitcast(
                            h4b[b, r, pl.ds(D // 2 + k * 16, 16)],
                            jnp.bfloat16)
                        p0, p1 = plsc.unpack(
                            va * vb, format=plsc.PackFormat.INTERLEAVED)
                        p = p + p0
                        q = q + p1
                    pbuf[j, pl.ds(0, 16)] = p + q
                acc = [plsc.load_gather(
                    pbuf, [iota16, jnp.full((16,), q, jnp.int32)])
                    for q in range(4)]
                for l in range(4, 16):
                    acc[l % 4] = acc[l % 4] + plsc.load_gather(
                        pbuf, [iota16, jnp.full((16,), l, jnp.int32)])
                scores = (acc[0] + acc[1]) + (acc[2] + acc[3])
                off = i * CH + g * 16
                idx16 = dsti[i, pl.ds(g * 16, 16)]
                a16 = plsc.load_gather(av, [idx16])
                c16 = plsc.load_gather(cv, [idx16])
                ea16 = eav[pl.ds(off, 16)]
                scores = (scores + ea16 * a16 + c16) * INV_SQRT_D
                exrow[b, 0, pl.ds(g * 16, 16)] = jnp.exp(scores)
                return carry
            lax.fori_loop(0, G, grp, 0)

        def drain_writes(i, b):
            pltpu.make_async_copy(exrow.at[b], ex_hbm.at[wid, i],
                                  sex[b]).wait()
            pltpu.make_async_copy(exrow.at[b, 0], dshared.at[dsti.at[i]],
                                  ssc[b]).wait()

        def finish_chunk(i, b):
            pltpu.async_copy(exrow.at[b], ex_hbm.at[wid, i], sex[b])
            pltpu.async_copy(exrow.at[b, 0], dshared.at[dsti.at[i]],
                             ssc[b], add=True)

        issue_gathers(0, 0)
        issue_gathers(1, 1)

        def triple(t, carry):
            for b in range(3):
                i = 3 * t + b
                issue_gathers(i + 2, (b + 2) % 3)

                @pl.when(t > 0)
                def _():
                    drain_writes(i, b)
                wait_gathers(i, b)
                compute_chunk(i, b)
                finish_chunk(i, b)
            return carry

        lax.fori_loop(0, (C - 2) // 3, triple, 0)
        drain_writes(C - 2, 0)
        wait_gathers(C - 2, 0)
        compute_chunk(C - 2, 0)
        finish_chunk(C - 2, 0)
        drain_writes(C - 1, 1)
        wait_gathers(C - 1, 1)
        compute_chunk(C - 1, 1)
        pltpu.sync_copy(exrow.at[1], ex_hbm.at[wid, C - 1])
        pltpu.sync_copy(exrow.at[1, 0], dshared.at[dsti.at[C - 1]], add=True)
        drain_writes(C - 2, 0)
        drain_writes(C - 3, 2)
        plsc.subcore_barrier()

        @pl.when(sid == 0)
        def _():
            pltpu.sync_copy(dshared, dpart_hbm.at[cid])

    return pl.kernel(
        body,
        out_type=[
            jax.ShapeDtypeStruct((NW, C, 1, CH), jnp.float32),
            jax.ShapeDtypeStruct((NC, N), jnp.float32),
        ],
        mesh=_sc_mesh(),
        compiler_params=pltpu.CompilerParams(needs_layout_passes=False),
        scratch_types=[
            pltpu.VMEM((C, CH), jnp.int32),
            pltpu.VMEM((C, CH), jnp.int32),
            pltpu.VMEM((EW,), jnp.float32),
            pltpu.VMEM((3, 1, CH), jnp.float32),
            pltpu.VMEM((N,), jnp.float32),
            pltpu.VMEM((N,), jnp.float32),
            pltpu.VMEM((3, CH, D), jnp.int32),
            pltpu.VMEM((3, CH, D), jnp.int32),
            pltpu.VMEM((16, 17), jnp.float32),
            pltpu.VMEM_SHARED((N,), jnp.float32),
        ] + [pltpu.SemaphoreType.DMA] * 12,
    )(src3, dst3, dst2, ea2, h3, h4, a, c)


def _aggregate_call(src4, dst3, ex4, h2):
    def body(src_hbm, dst_hbm, ex_hbm, h2_hbm, apart_hbm,
             dsti, srow, exread, h2b, ashared,
             sha, shb, ssa, ssb, sea, seb, sca, scb):
        cid = lax.axis_index("c")
        sid = lax.axis_index("s")
        wid = sid * NC + cid
        sh = (sha, shb)
        ss = (ssa, ssb)
        se = (sea, seb)
        sc = (sca, scb)

        pltpu.sync_copy(dst_hbm.at[wid], dsti)

        for r in range(CH):
            for k in range(D // 16):
                h2b[1, r, pl.ds(k * 16, 16)] = jnp.zeros((16,), jnp.float32)
        for t in range(7):
            pltpu.sync_copy(h2b.at[1],
                            ashared.at[pl.ds(sid * 624 + t * CH, CH)])
        pltpu.sync_copy(h2b.at[1, pl.ds(0, 64)],
                        ashared.at[pl.ds(sid * 624 + 560, 64)])

        @pl.when(sid == NS - 1)
        def _():
            pltpu.sync_copy(h2b.at[1, pl.ds(0, 16)],
                            ashared.at[pl.ds(9984, 16)])

        def issue_srow(i, b):
            pltpu.async_copy(src_hbm.at[wid, i], srow.at[b], ss[b])

        def wait_srow(i, b):
            pltpu.make_async_copy(src_hbm.at[wid, i], srow.at[b],
                                  ss[b]).wait()

        def issue_gather(b):
            pltpu.async_copy(h2_hbm.at[srow.at[b, 0]], h2b.at[b], sh[b])

        def wait_gather(b):
            pltpu.make_async_copy(h2_hbm.at[srow.at[b, 0]], h2b.at[b],
                                  sh[b]).wait()

        def issue_ex(i, b):
            pltpu.async_copy(ex_hbm.at[wid, i], exread.at[b], se[b])

        def wait_ex(i, b):
            pltpu.make_async_copy(ex_hbm.at[wid, i], exread.at[b],
                                  se[b]).wait()

        def scale_chunk(b):
            def grp(g, carry):
                w16 = exread[b, 0, pl.ds(g * 16, 16)]
                for j in range(16):
                    r = g * 16 + j
                    wj = w16[j]
                    for k in range(D // 16):
                        sl = pl.ds(k * 16, 16)
                        h2b[b, r, sl] = h2b[b, r, sl] * wj
                return carry
            lax.fori_loop(0, G, grp, 0)

        pltpu.sync_copy(src_hbm.at[wid, 0], srow.at[0])
        issue_gather(0)
        issue_ex(0, 0)
        issue_srow(1, 1)
        plsc.subcore_barrier()

        def wait_scatter(i, b):
            pltpu.make_async_copy(h2b.at[b], ashared.at[dsti.at[i]],
                                  sc[b]).wait()

        def pair(t, carry):
            for b in range(2):
                i = 2 * t + b
                wait_srow(i + 1, 1 - b)
                if b == 1:
                    wait_scatter(i - 1, 0)
                else:
                    @pl.when(t > 0)
                    def _():
                        wait_scatter(i - 1, 1)
                issue_gather(1 - b)
                issue_ex(i + 1, 1 - b)
                wait_gather(b)
                wait_ex(i, b)
                @pl.when(i + 2 <= C - 1)
                def _():
                    issue_srow(i + 2, b)
                scale_chunk(b)
                pltpu.async_copy(h2b.at[b], ashared.at[dsti.at[i]], sc[b],
                                 add=True)
            return carry

        lax.fori_loop(0, (C - 1) // 2, pair, 0)
        i_last = C - 1
        wait_scatter(i_last - 1, 1)
        wait_gather(0)
        wait_ex(i_last, 0)
        scale_chunk(0)
        pltpu.async_copy(h2b.at[0], ashared.at[dsti.at[i_last]], sc[0],
                         add=True)
        wait_scatter(i_last, 0)
        plsc.subcore_barrier()
        sl = pl.ds(sid * 624, 624)
        pltpu.sync_copy(ashared.at[sl], apart_hbm.at[cid, sl])

        @pl.when(sid == NS - 1)
        def _():
            sl2 = pl.ds(9984, 16)
            pltpu.sync_copy(ashared.at[sl2], apart_hbm.at[cid, sl2])

    return pl.kernel(
        body,
        out_type=jax.ShapeDtypeStruct((NC, N, D), jnp.float32),
        mesh=_sc_mesh(),
        compiler_params=pltpu.CompilerParams(needs_layout_passes=False),
        scratch_types=[
            pltpu.VMEM((C, CH), jnp.int32),
            pltpu.VMEM((2, 1, CH), jnp.int32),
            pltpu.VMEM((2, 1, CH), jnp.float32),
            pltpu.VMEM((2, CH, D), jnp.float32),
            pltpu.VMEM_SHARED((N, D), jnp.float32),
            pltpu.SemaphoreType.DMA,
            pltpu.SemaphoreType.DMA,
            pltpu.SemaphoreType.DMA,
            pltpu.SemaphoreType.DMA,
            pltpu.SemaphoreType.DMA,
            pltpu.SemaphoreType.DMA,
            pltpu.SemaphoreType.DMA,
            pltpu.SemaphoreType.DMA,
        ],
    )(src4, dst3, ex4, h2)


def _k4_body(b_ref, p_ref, q_ref, d0_ref, d1_ref, o_ref):
    den = d0_ref[...] + d1_ref[...] + 1e-16
    o_ref[...] = b_ref[...] + (p_ref[...] + q_ref[...]) / den


def _combine(base, ap0, ap1, d0, d1):
    BN = 1000
    spec = pl.BlockSpec((BN, D), lambda i: (i, 0))
    v_spec = pl.BlockSpec((BN, 1), lambda i: (i, 0))
    return pl.pallas_call(
        _k4_body,
        grid=(N // BN,),
        in_specs=[spec, spec, spec, v_spec, v_spec],
        out_specs=spec,
        out_shape=jax.ShapeDtypeStruct((N, D), jnp.float32),
    )(base, ap0, ap1, d0, d1)


def kernel(x, edge_index, edge_attr, W1, b1, W2, b2, W3, b3, W4, b4, W5, b5):
    src = edge_index[0]
    dst = edge_index[1]
    src3 = src.reshape(NW, C, CH)
    dst3 = dst.reshape(NW, C, CH)
    dst2 = dst.reshape(NW, EW)
    ea2 = edge_attr.reshape(NW, EW)
    w5 = W5[:, 0]

    base, h2, h3, h4, a, c = _node_transforms(
        x, W1, b1, W2, b2, W3, b3, W4, b4, w5, b5)
    h3i = lax.bitcast_convert_type(h3.reshape(N, D // 2, 2), jnp.int32)
    h4i = lax.bitcast_convert_type(h4.reshape(N, D // 2, 2), jnp.int32)
    z = jnp.concatenate([h3i, h4i], axis=1)
    ex4, dpart = _edge_scores_call(
        src3, dst3, dst2, ea2, z, z, a.reshape(N), c.reshape(N))
    apart = _aggregate_call(src.reshape(NW, C, 1, CH), dst3, ex4, h2)
    return _combine(base, apart[0], apart[1],
                    dpart[0].reshape(N, 1), dpart[1].reshape(N, 1))


# speedup vs baseline: 1.0874x; 1.0874x over previous
import math

import jax
import jax.numpy as jnp
from jax import lax
from jax.experimental import pallas as pl
from jax.experimental.pallas import tpu as pltpu
from jax.experimental.pallas import tpu_sc as plsc

N = 10000
E = 320000
D = 128

NC = 2
NS = 16
NW = NC * NS
EW = E // NW
CH = 80
C = EW // CH
G = CH // 16
INV_SQRT_D = 1.0 / math.sqrt(float(D))


def _k1_body(x_ref, w1, b1, w2, b2, w3, b3, w4, b4, w5, b5,
             base_ref, h2_ref, h3_ref, h4_ref, a_ref, c_ref):
    xb = x_ref[...]
    dot = lambda m, w: lax.dot_general(
        m, w[...], (((1,), (1,)), ((), ())), preferred_element_type=jnp.float32)
    base_ref[...] = dot(xb, w1) + b1[...][None, :]
    h2_ref[...] = dot(xb, w2) + b2[...][None, :]
    h3 = dot(xb, w3) + b3[...][None, :]
    h3_ref[...] = h3
    h4_ref[...] = dot(xb, w4) + b4[...][None, :]
    a_ref[...] = jnp.sum(h3 * w5[...][None, :], axis=1, keepdims=True)
    c_ref[...] = jnp.sum(h3 * b5[...][None, :], axis=1, keepdims=True)


def _node_transforms(x, W1, b1, W2, b2, W3, b3, W4, b4, w5, b5):
    BN = 1000
    row_spec = pl.BlockSpec((BN, D), lambda i: (i, 0))
    w_spec = pl.BlockSpec((D, D), lambda i: (0, 0))
    b_spec = pl.BlockSpec((D,), lambda i: (0,))
    v_spec = pl.BlockSpec((BN, 1), lambda i: (i, 0))
    return pl.pallas_call(
        _k1_body,
        grid=(N // BN,),
        in_specs=[row_spec, w_spec, b_spec, w_spec, b_spec, w_spec, b_spec,
                  w_spec, b_spec, b_spec, b_spec],
        out_specs=[row_spec, row_spec, row_spec, row_spec, v_spec, v_spec],
        out_shape=[
            jax.ShapeDtypeStruct((N, D), jnp.float32),
            jax.ShapeDtypeStruct((N, D), jnp.float32),
            jax.ShapeDtypeStruct((N, D), jnp.float32),
            jax.ShapeDtypeStruct((N, D), jnp.float32),
            jax.ShapeDtypeStruct((N, 1), jnp.float32),
            jax.ShapeDtypeStruct((N, 1), jnp.float32),
        ],
    )(x, W1, b1, W2, b2, W3, b3, W4, b4, w5, b5)


def _sc_mesh():
    return plsc.VectorSubcoreMesh(
        core_axis_name="c", subcore_axis_name="s", num_cores=NC,
        num_subcores=NS)


def _edge_scores_call(src3, dst3, dst2, ea2, h3, h4, a, c):
    def body(src_hbm, dst_hbm, dstf_hbm, ea_hbm, h3_hbm, h4_hbm, a_hbm, c_hbm,
             ex_hbm, dpart_hbm,
             srci, dsti, eav, exrow, av, cv, h3b, h4b, pbuf, dshared,
             s3a, s3b, s3c, s4a, s4b, s4c, sexa, sexb, sexc,
             ssca, sscb, sscc):
        iota16 = lax.iota(jnp.int32, 16)
        cid = lax.axis_index("c")
        sid = lax.axis_index("s")
        wid = sid * NC + cid
        s3 = (s3a, s3b, s3c)
        s4 = (s4a, s4b, s4c)
        sex = (sexa, sexb, sexc)
        ssc = (ssca, sscb, sscc)

        @pl.when(sid == 0)
        def _():
            def zb(i, carry):
                av[pl.ds(i * 16, 16)] = jnp.zeros((16,), jnp.float32)
                return carry
            lax.fori_loop(0, N // 16, zb, 0)
            pltpu.sync_copy(av, dshared)

        pltpu.sync_copy(src_hbm.at[wid], srci)
        pltpu.sync_copy(dst_hbm.at[wid], dsti)
        pltpu.sync_copy(ea_hbm.at[wid], eav)
        pltpu.sync_copy(a_hbm, av)
        pltpu.sync_copy(c_hbm, cv)
        plsc.subcore_barrier()

        def issue_gathers(i, b):
            pltpu.async_copy(h3_hbm.at[dsti.at[i]], h3b.at[b], s3[b])
            pltpu.async_copy(h4_hbm.at[srci.at[i]], h4b.at[b], s4[b])

        def wait_gathers(i, b):
            pltpu.make_async_copy(h3_hbm.at[dsti.at[i]], h3b.at[b],
                                  s3[b]).wait()
            pltpu.make_async_copy(h4_hbm.at[srci.at[i]], h4b.at[b],
                                  s4[b]).wait()

        def compute_chunk(i, b):
            def grp(g, carry):
                for j in range(16):
                    r = g * 16 + j
                    p = h3b[b, r, pl.ds(0, 16)] * h4b[b, r, pl.ds(0, 16)]
                    for k in range(1, 8):
                        p = p + (h3b[b, r, pl.ds(k * 16, 16)] *
                                 h4b[b, r, pl.ds(k * 16, 16)])
                    pbuf[j, pl.ds(0, 16)] = p
                acc = [plsc.load_gather(
                    pbuf, [iota16, jnp.full((16,), q, jnp.int32)])
                    for q in range(4)]
                for l in range(4, 16):
                    acc[l % 4] = acc[l % 4] + plsc.load_gather(
                        pbuf, [iota16, jnp.full((16,), l, jnp.int32)])
                scores = (acc[0] + acc[1]) + (acc[2] + acc[3])
                off = i * CH + g * 16
                idx16 = dsti[i, pl.ds(g * 16, 16)]
                a16 = plsc.load_gather(av, [idx16])
                c16 = plsc.load_gather(cv, [idx16])
                ea16 = eav[pl.ds(off, 16)]
                scores = (scores + ea16 * a16 + c16) * INV_SQRT_D
                exrow[b, 0, pl.ds(g * 16, 16)] = jnp.exp(scores)
                return carry
            lax.fori_loop(0, G, grp, 0)

        def drain_writes(i, b):
            pltpu.make_async_copy(exrow.at[b], ex_hbm.at[wid, i],
                                  sex[b]).wait()
            pltpu.make_async_copy(exrow.at[b, 0], dshared.at[dsti.at[i]],
                                  ssc[b]).wait()

        def finish_chunk(i, b):
            pltpu.async_copy(exrow.at[b], ex_hbm.at[wid, i], sex[b])
            pltpu.async_copy(exrow.at[b, 0], dshared.at[dsti.at[i]],
                             ssc[b], add=True)

        issue_gathers(0, 0)
        issue_gathers(1, 1)

        def triple(t, carry):
            for b in range(3):
                i = 3 * t + b
                issue_gathers(i + 2, (b + 2) % 3)

                @pl.when(t > 0)
                def _():
                    drain_writes(i, b)
                wait_gathers(i, b)
                compute_chunk(i, b)
                finish_chunk(i, b)
            return carry

        lax.fori_loop(0, (C - 2) // 3, triple, 0)
        drain_writes(C - 2, 0)
        wait_gathers(C - 2, 0)
        compute_chunk(C - 2, 0)
        finish_chunk(C - 2, 0)
        drain_writes(C - 1, 1)
        wait_gathers(C - 1, 1)
        compute_chunk(C - 1, 1)
        pltpu.sync_copy(exrow.at[1], ex_hbm.at[wid, C - 1])
        pltpu.sync_copy(exrow.at[1, 0], dshared.at[dsti.at[C - 1]], add=True)
        drain_writes(C - 2, 0)
        drain_writes(C - 3, 2)
        plsc.subcore_barrier()

        @pl.when(sid == 0)
        def _():
            pltpu.sync_copy(dshared, dpart_hbm.at[cid])

    return pl.kernel(
        body,
        out_type=[
            jax.ShapeDtypeStruct((NW, C, 1, CH), jnp.float32),
            jax.ShapeDtypeStruct((NC, N), jnp.float32),
        ],
        mesh=_sc_mesh(),
        compiler_params=pltpu.CompilerParams(needs_layout_passes=False),
        scratch_types=[
            pltpu.VMEM((C, CH), jnp.int32),
            pltpu.VMEM((C, CH), jnp.int32),
            pltpu.VMEM((EW,), jnp.float32),
            pltpu.VMEM((3, 1, CH), jnp.float32),
            pltpu.VMEM((N,), jnp.float32),
            pltpu.VMEM((N,), jnp.float32),
            pltpu.VMEM((3, CH, D), jnp.float32),
            pltpu.VMEM((3, CH, D), jnp.float32),
            pltpu.VMEM((16, 17), jnp.float32),
            pltpu.VMEM_SHARED((N,), jnp.float32),
        ] + [pltpu.SemaphoreType.DMA] * 12,
    )(src3, dst3, dst2, ea2, h3, h4, a, c)


def _aggregate_call(src4, dst3, ex4, h2):
    def body(src_hbm, dst_hbm, ex_hbm, h2_hbm, apart_hbm,
             dsti, srow, exread, h2b, ashared,
             sha, shb, ssa, ssb, sea, seb, sca, scb):
        cid = lax.axis_index("c")
        sid = lax.axis_index("s")
        wid = sid * NC + cid
        sh = (sha, shb)
        ss = (ssa, ssb)
        se = (sea, seb)
        sc = (sca, scb)

        pltpu.sync_copy(dst_hbm.at[wid], dsti)

        for r in range(CH):
            for k in range(D // 16):
                h2b[1, r, pl.ds(k * 16, 16)] = jnp.zeros((16,), jnp.float32)
        for t in range(7):
            pltpu.sync_copy(h2b.at[1],
                            ashared.at[pl.ds(sid * 624 + t * CH, CH)])
        pltpu.sync_copy(h2b.at[1, pl.ds(0, 64)],
                        ashared.at[pl.ds(sid * 624 + 560, 64)])

        @pl.when(sid == NS - 1)
        def _():
            pltpu.sync_copy(h2b.at[1, pl.ds(0, 16)],
                            ashared.at[pl.ds(9984, 16)])

        def issue_srow(i, b):
            pltpu.async_copy(src_hbm.at[wid, i], srow.at[b], ss[b])

        def wait_srow(i, b):
            pltpu.make_async_copy(src_hbm.at[wid, i], srow.at[b],
                                  ss[b]).wait()

        def issue_gather(b):
            pltpu.async_copy(h2_hbm.at[srow.at[b, 0]], h2b.at[b], sh[b])

        def wait_gather(b):
            pltpu.make_async_copy(h2_hbm.at[srow.at[b, 0]], h2b.at[b],
                                  sh[b]).wait()

        def issue_ex(i, b):
            pltpu.async_copy(ex_hbm.at[wid, i], exread.at[b], se[b])

        def wait_ex(i, b):
            pltpu.make_async_copy(ex_hbm.at[wid, i], exread.at[b],
                                  se[b]).wait()

        def scale_chunk(b):
            def grp(g, carry):
                w16 = exread[b, 0, pl.ds(g * 16, 16)]
                for j in range(16):
                    r = g * 16 + j
                    wj = w16[j]
                    for k in range(D // 16):
                        sl = pl.ds(k * 16, 16)
                        h2b[b, r, sl] = h2b[b, r, sl] * wj
                return carry
            lax.fori_loop(0, G, grp, 0)

        pltpu.sync_copy(src_hbm.at[wid, 0], srow.at[0])
        issue_gather(0)
        issue_ex(0, 0)
        issue_srow(1, 1)
        plsc.subcore_barrier()

        def wait_scatter(i, b):
            pltpu.make_async_copy(h2b.at[b], ashared.at[dsti.at[i]],
                                  sc[b]).wait()

        def pair(t, carry):
            for b in range(2):
                i = 2 * t + b
                wait_srow(i + 1, 1 - b)
                if b == 1:
                    wait_scatter(i - 1, 0)
                else:
                    @pl.when(t > 0)
                    def _():
                        wait_scatter(i - 1, 1)
                issue_gather(1 - b)
                issue_ex(i + 1, 1 - b)
                wait_gather(b)
                wait_ex(i, b)
                @pl.when(i + 2 <= C - 1)
                def _():
                    issue_srow(i + 2, b)
                scale_chunk(b)
                pltpu.async_copy(h2b.at[b], ashared.at[dsti.at[i]], sc[b],
                                 add=True)
            return carry

        lax.fori_loop(0, (C - 1) // 2, pair, 0)
        i_last = C - 1
        wait_scatter(i_last - 1, 1)
        wait_gather(0)
        wait_ex(i_last, 0)
        scale_chunk(0)
        pltpu.async_copy(h2b.at[0], ashared.at[dsti.at[i_last]], sc[0],
                         add=True)
        wait_scatter(i_last, 0)
        plsc.subcore_barrier()
        sl = pl.ds(sid * 624, 624)
        pltpu.sync_copy(ashared.at[sl], apart_hbm.at[cid, sl])

        @pl.when(sid == NS - 1)
        def _():
            sl2 = pl.ds(9984, 16)
            pltpu.sync_copy(ashared.at[sl2], apart_hbm.at[cid, sl2])

    return pl.kernel(
        body,
        out_type=jax.ShapeDtypeStruct((NC, N, D), jnp.float32),
        mesh=_sc_mesh(),
        compiler_params=pltpu.CompilerParams(needs_layout_passes=False),
        scratch_types=[
            pltpu.VMEM((C, CH), jnp.int32),
            pltpu.VMEM((2, 1, CH), jnp.int32),
            pltpu.VMEM((2, 1, CH), jnp.float32),
            pltpu.VMEM((2, CH, D), jnp.float32),
            pltpu.VMEM_SHARED((N, D), jnp.float32),
            pltpu.SemaphoreType.DMA,
            pltpu.SemaphoreType.DMA,
            pltpu.SemaphoreType.DMA,
            pltpu.SemaphoreType.DMA,
            pltpu.SemaphoreType.DMA,
            pltpu.SemaphoreType.DMA,
            pltpu.SemaphoreType.DMA,
            pltpu.SemaphoreType.DMA,
        ],
    )(src4, dst3, ex4, h2)


def _k4_body(b_ref, p_ref, q_ref, d0_ref, d1_ref, o_ref):
    den = d0_ref[...] + d1_ref[...] + 1e-16
    o_ref[...] = b_ref[...] + (p_ref[...] + q_ref[...]) / den


def _combine(base, ap0, ap1, d0, d1):
    BN = 1000
    spec = pl.BlockSpec((BN, D), lambda i: (i, 0))
    v_spec = pl.BlockSpec((BN, 1), lambda i: (i, 0))
    return pl.pallas_call(
        _k4_body,
        grid=(N // BN,),
        in_specs=[spec, spec, spec, v_spec, v_spec],
        out_specs=spec,
        out_shape=jax.ShapeDtypeStruct((N, D), jnp.float32),
    )(base, ap0, ap1, d0, d1)


def kernel(x, edge_index, edge_attr, W1, b1, W2, b2, W3, b3, W4, b4, W5, b5):
    src = edge_index[0]
    dst = edge_index[1]
    src3 = src.reshape(NW, C, CH)
    dst3 = dst.reshape(NW, C, CH)
    dst2 = dst.reshape(NW, EW)
    ea2 = edge_attr.reshape(NW, EW)
    w5 = W5[:, 0]

    base, h2, h3, h4, a, c = _node_transforms(
        x, W1, b1, W2, b2, W3, b3, W4, b4, w5, b5)
    ex4, dpart = _edge_scores_call(
        src3, dst3, dst2, ea2, h3, h4, a.reshape(N), c.reshape(N))
    apart = _aggregate_call(src.reshape(NW, C, 1, CH), dst3, ex4, h2)
    return _combine(base, apart[0], apart[1],
                    dpart[0].reshape(N, 1), dpart[1].reshape(N, 1))


# speedup vs baseline: 1.0946x; 1.0066x over previous
import math

import jax
import jax.numpy as jnp
from jax import lax
from jax.experimental import pallas as pl
from jax.experimental.pallas import tpu as pltpu
from jax.experimental.pallas import tpu_sc as plsc

N = 10000
E = 320000
D = 128

NC = 2
NS = 16
NW = NC * NS
EW = E // NW
CH = 80
C = EW // CH
G = CH // 16
INV_SQRT_D = 1.0 / math.sqrt(float(D))


def _k1_body(x_ref, w1, b1, w2, b2, w3, b3, w4, b4, w5, b5,
             base_ref, h2_ref, h3_ref, h4_ref, a_ref, c_ref):
    xb = x_ref[...]
    dot = lambda m, w: lax.dot_general(
        m, w[...], (((1,), (1,)), ((), ())), preferred_element_type=jnp.float32)
    base_ref[...] = dot(xb, w1) + b1[...][None, :]
    h2_ref[...] = dot(xb, w2) + b2[...][None, :]
    h3 = dot(xb, w3) + b3[...][None, :]
    h3_ref[...] = h3
    h4_ref[...] = dot(xb, w4) + b4[...][None, :]
    a_ref[...] = jnp.sum(h3 * w5[...][None, :], axis=1, keepdims=True)
    c_ref[...] = jnp.sum(h3 * b5[...][None, :], axis=1, keepdims=True)


def _node_transforms(x, W1, b1, W2, b2, W3, b3, W4, b4, w5, b5):
    BN = 1000
    row_spec = pl.BlockSpec((BN, D), lambda i: (i, 0))
    w_spec = pl.BlockSpec((D, D), lambda i: (0, 0))
    b_spec = pl.BlockSpec((D,), lambda i: (0,))
    v_spec = pl.BlockSpec((BN, 1), lambda i: (i, 0))
    return pl.pallas_call(
        _k1_body,
        grid=(N // BN,),
        in_specs=[row_spec, w_spec, b_spec, w_spec, b_spec, w_spec, b_spec,
                  w_spec, b_spec, b_spec, b_spec],
        out_specs=[row_spec, row_spec, row_spec, row_spec, v_spec, v_spec],
        out_shape=[
            jax.ShapeDtypeStruct((N, D), jnp.float32),
            jax.ShapeDtypeStruct((N, D), jnp.float32),
            jax.ShapeDtypeStruct((N, D), jnp.float32),
            jax.ShapeDtypeStruct((N, D), jnp.float32),
            jax.ShapeDtypeStruct((N, 1), jnp.float32),
            jax.ShapeDtypeStruct((N, 1), jnp.float32),
        ],
    )(x, W1, b1, W2, b2, W3, b3, W4, b4, w5, b5)


def _sc_mesh():
    return plsc.VectorSubcoreMesh(
        core_axis_name="c", subcore_axis_name="s", num_cores=NC,
        num_subcores=NS)


def _edge_scores_call(src3, dst3, ea2, h3, h4, a, c):
    def body(src_hbm, dst_hbm, ea_hbm, h3_hbm, h4_hbm, a_hbm, c_hbm,
             ex_hbm, dpart_hbm,
             srci, dsti, eav, exrow, av, cv, h3b, h4b, pbuf, dshared,
             s3a, s3b, s3c, s4a, s4b, s4c, sexa, sexb, sexc,
             ssca, sscb, sscc):
        iota16 = lax.iota(jnp.int32, 16)
        cid = lax.axis_index("c")
        sid = lax.axis_index("s")
        wid = sid * NC + cid
        s3 = (s3a, s3b, s3c)
        s4 = (s4a, s4b, s4c)
        sex = (sexa, sexb, sexc)
        ssc = (ssca, sscb, sscc)

        @pl.when(sid == 0)
        def _():
            def zb(i, carry):
                av[pl.ds(i * 16, 16)] = jnp.zeros((16,), jnp.float32)
                return carry
            lax.fori_loop(0, N // 16, zb, 0)
            pltpu.sync_copy(av, dshared)

        pltpu.sync_copy(src_hbm.at[wid], srci)
        pltpu.sync_copy(dst_hbm.at[wid], dsti)
        pltpu.sync_copy(ea_hbm.at[wid], eav)
        pltpu.sync_copy(a_hbm, av)
        pltpu.sync_copy(c_hbm, cv)
        plsc.subcore_barrier()

        def issue_gathers(i, b):
            pltpu.async_copy(h3_hbm.at[dsti.at[i]], h3b.at[b], s3[b])
            pltpu.async_copy(h4_hbm.at[srci.at[i]], h4b.at[b], s4[b])

        def wait_gathers(i, b):
            pltpu.make_async_copy(h3_hbm.at[dsti.at[i]], h3b.at[b],
                                  s3[b]).wait()
            pltpu.make_async_copy(h4_hbm.at[srci.at[i]], h4b.at[b],
                                  s4[b]).wait()

        def compute_chunk(i, b):
            def grp(g, carry):
                for j in range(16):
                    r = g * 16 + j
                    p = h3b[b, r, pl.ds(0, 16)] * h4b[b, r, pl.ds(0, 16)]
                    for k in range(1, 8):
                        p = p + (h3b[b, r, pl.ds(k * 16, 16)] *
                                 h4b[b, r, pl.ds(k * 16, 16)])
                    pbuf[j, pl.ds(0, 16)] = p
                acc = [plsc.load_gather(
                    pbuf, [iota16, jnp.full((16,), q, jnp.int32)])
                    for q in range(4)]
                for l in range(4, 16):
                    acc[l % 4] = acc[l % 4] + plsc.load_gather(
                        pbuf, [iota16, jnp.full((16,), l, jnp.int32)])
                scores = (acc[0] + acc[1]) + (acc[2] + acc[3])
                off = i * CH + g * 16
                idx16 = dsti[i, pl.ds(g * 16, 16)]
                a16 = plsc.load_gather(av, [idx16])
                c16 = plsc.load_gather(cv, [idx16])
                ea16 = eav[pl.ds(off, 16)]
                scores = (scores + ea16 * a16 + c16) * INV_SQRT_D
                exrow[b, 0, pl.ds(g * 16, 16)] = jnp.exp(scores)
                return carry
            lax.fori_loop(0, G, grp, 0)

        def drain_writes(i, b):
            pltpu.make_async_copy(exrow.at[b], ex_hbm.at[wid, i],
                                  sex[b]).wait()
            pltpu.make_async_copy(exrow.at[b, 0], dshared.at[dsti.at[i]],
                                  ssc[b]).wait()

        def finish_chunk(i, b):
            pltpu.async_copy(exrow.at[b], ex_hbm.at[wid, i], sex[b])
            pltpu.async_copy(exrow.at[b, 0], dshared.at[dsti.at[i]],
                             ssc[b], add=True)

        issue_gathers(0, 0)
        issue_gathers(1, 1)

        def triple(t, carry):
            for b in range(3):
                i = 3 * t + b
                issue_gathers(i + 2, (b + 2) % 3)

                @pl.when(t > 0)
                def _():
                    drain_writes(i, b)
                wait_gathers(i, b)
                compute_chunk(i, b)
                finish_chunk(i, b)
            return carry

        lax.fori_loop(0, (C - 2) // 3, triple, 0)
        drain_writes(C - 2, 0)
        wait_gathers(C - 2, 0)
        compute_chunk(C - 2, 0)
        finish_chunk(C - 2, 0)
        drain_writes(C - 1, 1)
        wait_gathers(C - 1, 1)
        compute_chunk(C - 1, 1)
        pltpu.sync_copy(exrow.at[1], ex_hbm.at[wid, C - 1])
        pltpu.sync_copy(exrow.at[1, 0], dshared.at[dsti.at[C - 1]], add=True)
        drain_writes(C - 2, 0)
        drain_writes(C - 3, 2)
        plsc.subcore_barrier()

        @pl.when(sid == 0)
        def _():
            pltpu.sync_copy(dshared, dpart_hbm.at[cid])

    return pl.kernel(
        body,
        out_type=[
            jax.ShapeDtypeStruct((NW, C, 1, CH), jnp.float32),
            jax.ShapeDtypeStruct((NC, N), jnp.float32),
        ],
        mesh=_sc_mesh(),
        compiler_params=pltpu.CompilerParams(needs_layout_passes=False),
        scratch_types=[
            pltpu.VMEM((C, CH), jnp.int32),
            pltpu.VMEM((C, CH), jnp.int32),
            pltpu.VMEM((EW,), jnp.float32),
            pltpu.VMEM((3, 1, CH), jnp.float32),
            pltpu.VMEM((N,), jnp.float32),
            pltpu.VMEM((N,), jnp.float32),
            pltpu.VMEM((3, CH, D), jnp.float32),
            pltpu.VMEM((3, CH, D), jnp.float32),
            pltpu.VMEM((16, 17), jnp.float32),
            pltpu.VMEM_SHARED((N,), jnp.float32),
        ] + [pltpu.SemaphoreType.DMA] * 12,
    )(src3, dst3, ea2, h3, h4, a, c)


def _aggregate_call(src4, dst3, ex4, h2):
    def body(src_hbm, dst_hbm, ex_hbm, h2_hbm, apart_hbm,
             dsti, srow, exread, h2b, ashared,
             sha, shb, ssa, ssb, sea, seb, sca, scb):
        cid = lax.axis_index("c")
        sid = lax.axis_index("s")
        wid = sid * NC + cid
        sh = (sha, shb)
        ss = (ssa, ssb)
        se = (sea, seb)
        sc = (sca, scb)

        pltpu.sync_copy(dst_hbm.at[wid], dsti)

        for r in range(CH):
            for k in range(D // 16):
                h2b[1, r, pl.ds(k * 16, 16)] = jnp.zeros((16,), jnp.float32)
        for t in range(7):
            pltpu.sync_copy(h2b.at[1],
                            ashared.at[pl.ds(sid * 624 + t * CH, CH)])
        pltpu.sync_copy(h2b.at[1, pl.ds(0, 64)],
                        ashared.at[pl.ds(sid * 624 + 560, 64)])

        @pl.when(sid == NS - 1)
        def _():
            pltpu.sync_copy(h2b.at[1, pl.ds(0, 16)],
                            ashared.at[pl.ds(9984, 16)])

        def issue_srow(i, b):
            pltpu.async_copy(src_hbm.at[wid, i], srow.at[b], ss[b])

        def wait_srow(i, b):
            pltpu.make_async_copy(src_hbm.at[wid, i], srow.at[b],
                                  ss[b]).wait()

        def issue_gather(b):
            pltpu.async_copy(h2_hbm.at[srow.at[b, 0]], h2b.at[b], sh[b])

        def wait_gather(b):
            pltpu.make_async_copy(h2_hbm.at[srow.at[b, 0]], h2b.at[b],
                                  sh[b]).wait()

        def issue_ex(i, b):
            pltpu.async_copy(ex_hbm.at[wid, i], exread.at[b], se[b])

        def wait_ex(i, b):
            pltpu.make_async_copy(ex_hbm.at[wid, i], exread.at[b],
                                  se[b]).wait()

        def scale_chunk(b):
            def grp(g, carry):
                w16 = exread[b, 0, pl.ds(g * 16, 16)]
                for j in range(16):
                    r = g * 16 + j
                    wj = w16[j]
                    for k in range(D // 16):
                        sl = pl.ds(k * 16, 16)
                        h2b[b, r, sl] = h2b[b, r, sl] * wj
                return carry
            lax.fori_loop(0, G, grp, 0)

        pltpu.sync_copy(src_hbm.at[wid, 0], srow.at[0])
        issue_gather(0)
        issue_ex(0, 0)
        issue_srow(1, 1)
        plsc.subcore_barrier()

        def wait_scatter(i, b):
            pltpu.make_async_copy(h2b.at[b], ashared.at[dsti.at[i]],
                                  sc[b]).wait()

        def pair(t, carry):
            for b in range(2):
                i = 2 * t + b
                wait_srow(i + 1, 1 - b)
                if b == 1:
                    wait_scatter(i - 1, 0)
                else:
                    @pl.when(t > 0)
                    def _():
                        wait_scatter(i - 1, 1)
                issue_gather(1 - b)
                issue_ex(i + 1, 1 - b)
                wait_gather(b)
                wait_ex(i, b)
                @pl.when(i + 2 <= C - 1)
                def _():
                    issue_srow(i + 2, b)
                scale_chunk(b)
                pltpu.async_copy(h2b.at[b], ashared.at[dsti.at[i]], sc[b],
                                 add=True)
            return carry

        lax.fori_loop(0, (C - 1) // 2, pair, 0)
        i_last = C - 1
        wait_scatter(i_last - 1, 1)
        wait_gather(0)
        wait_ex(i_last, 0)
        scale_chunk(0)
        pltpu.async_copy(h2b.at[0], ashared.at[dsti.at[i_last]], sc[0],
                         add=True)
        wait_scatter(i_last, 0)
        plsc.subcore_barrier()
        sl = pl.ds(sid * 624, 624)
        pltpu.sync_copy(ashared.at[sl], apart_hbm.at[cid, sl])

        @pl.when(sid == NS - 1)
        def _():
            sl2 = pl.ds(9984, 16)
            pltpu.sync_copy(ashared.at[sl2], apart_hbm.at[cid, sl2])

    return pl.kernel(
        body,
        out_type=jax.ShapeDtypeStruct((NC, N, D), jnp.float32),
        mesh=_sc_mesh(),
        compiler_params=pltpu.CompilerParams(needs_layout_passes=False),
        scratch_types=[
            pltpu.VMEM((C, CH), jnp.int32),
            pltpu.VMEM((2, 1, CH), jnp.int32),
            pltpu.VMEM((2, 1, CH), jnp.float32),
            pltpu.VMEM((2, CH, D), jnp.float32),
            pltpu.VMEM_SHARED((N, D), jnp.float32),
            pltpu.SemaphoreType.DMA,
            pltpu.SemaphoreType.DMA,
            pltpu.SemaphoreType.DMA,
            pltpu.SemaphoreType.DMA,
            pltpu.SemaphoreType.DMA,
            pltpu.SemaphoreType.DMA,
            pltpu.SemaphoreType.DMA,
            pltpu.SemaphoreType.DMA,
        ],
    )(src4, dst3, ex4, h2)


def _k4_body(b_ref, p_ref, q_ref, d0_ref, d1_ref, o_ref):
    den = d0_ref[...] + d1_ref[...] + 1e-16
    o_ref[...] = b_ref[...] + (p_ref[...] + q_ref[...]) / den


def _combine(base, ap0, ap1, d0, d1):
    BN = 1000
    spec = pl.BlockSpec((BN, D), lambda i: (i, 0))
    v_spec = pl.BlockSpec((BN, 1), lambda i: (i, 0))
    return pl.pallas_call(
        _k4_body,
        grid=(N // BN,),
        in_specs=[spec, spec, spec, v_spec, v_spec],
        out_specs=spec,
        out_shape=jax.ShapeDtypeStruct((N, D), jnp.float32),
    )(base, ap0, ap1, d0, d1)


def kernel(x, edge_index, edge_attr, W1, b1, W2, b2, W3, b3, W4, b4, W5, b5):
    src = edge_index[0]
    dst = edge_index[1]
    src3 = src.reshape(NW, C, CH)
    dst3 = dst.reshape(NW, C, CH)
    ea2 = edge_attr.reshape(NW, EW)
    w5 = W5[:, 0]

    base, h2, h3, h4, a, c = _node_transforms(
        x, W1, b1, W2, b2, W3, b3, W4, b4, w5, b5)
    ex4, dpart = _edge_scores_call(
        src3, dst3, ea2, h3, h4, a.reshape(N), c.reshape(N))
    apart = _aggregate_call(src.reshape(NW, C, 1, CH), dst3, ex4, h2)
    return _combine(base, apart[0], apart[1],
                    dpart[0].reshape(N, 1), dpart[1].reshape(N, 1))
